# baseline (speedup 1.0000x reference)
import jax
import jax.numpy as jnp
from jax.experimental import pallas as pl
from jax.experimental.pallas import tpu as pltpu

_B = 8
_N = 2048
_BLK = 256


def _copy_body(cov_hbm, out_hbm, *sems):
    copies = [
        pltpu.make_async_copy(cov_hbm.at[b], out_hbm.at[b], sems[b])
        for b in range(_B)
    ]
    for c in copies:
        c.start()
    for c in copies:
        c.wait()


def _diag_kernel(emb_ref, ns_ref, cov_ref, out_ref):
    ev = jnp.exp(emb_ref[0]) + jnp.exp(ns_ref[0, 0])
    row = jax.lax.broadcasted_iota(jnp.int32, (_BLK, _BLK), 0)
    col = jax.lax.broadcasted_iota(jnp.int32, (_BLK, _BLK), 1)
    out_ref[0] = cov_ref[0] + jnp.where(row == col, ev, 0.0)


def kernel(cov, embeddings, noise_scale):
    emb = embeddings[:, :, -1].reshape(_B, 1, _N)
    ns = noise_scale.reshape(1, 1)
    copied = pl.pallas_call(
        _copy_body,
        in_specs=[pl.BlockSpec(memory_space=pl.ANY)],
        out_specs=pl.BlockSpec(memory_space=pl.ANY),
        out_shape=jax.ShapeDtypeStruct((_B, _N, _N), jnp.float32),
        scratch_shapes=[pltpu.SemaphoreType.DMA] * _B,
    )(cov)
    return pl.pallas_call(
        _diag_kernel,
        grid=(_B, _N // _BLK),
        in_specs=[
            pl.BlockSpec((1, 1, _BLK), lambda b, i: (b, 0, i)),
            pl.BlockSpec((1, 1), lambda b, i: (0, 0)),
            pl.BlockSpec((1, _BLK, _BLK), lambda b, i: (b, i, i)),
        ],
        out_specs=pl.BlockSpec((1, _BLK, _BLK), lambda b, i: (b, i, i)),
        out_shape=jax.ShapeDtypeStruct((_B, _N, _N), jnp.float32),
        input_output_aliases={2: 0},
    )(emb, ns, copied)


# speedup vs baseline: 47.8044x; 47.8044x over previous
import jax
import jax.numpy as jnp
from jax.experimental import pallas as pl
from jax.experimental.pallas import tpu as pltpu

_B = 8
_N = 2048
_CH = 512
_PER_B = _N // _CH
_CHUNKS = _B * _PER_B
_NBUF = 6


def _body(emb_ref, ns_ref, cov_hbm, out_hbm, buf, in_sems, out_sems):
    def in_copy(k):
        b, r0 = k // _PER_B, (k % _PER_B) * _CH
        return pltpu.make_async_copy(
            cov_hbm.at[b, pl.ds(r0, _CH)], buf.at[k % _NBUF], in_sems.at[k % _NBUF]
        )

    def out_copy(k):
        b, r0 = k // _PER_B, (k % _PER_B) * _CH
        return pltpu.make_async_copy(
            buf.at[k % _NBUF], out_hbm.at[b, pl.ds(r0, _CH)], out_sems.at[k % _NBUF]
        )

    row = jax.lax.broadcasted_iota(jnp.int32, (_CH, _CH), 0)
    col = jax.lax.broadcasted_iota(jnp.int32, (_CH, _CH), 1)
    ns = jnp.exp(ns_ref[0, 0])

    for j in range(min(_NBUF, _CHUNKS)):
        in_copy(j).start()

    waited_out = set()
    for k in range(_CHUNKS):
        b, r0 = k // _PER_B, (k % _PER_B) * _CH
        in_copy(k).wait()
        ev = jnp.exp(emb_ref[b, :, pl.ds(r0, _CH)]) + ns
        i = k % _NBUF
        buf[i, :, pl.ds(r0, _CH)] = buf[i, :, pl.ds(r0, _CH)] + jnp.where(
            row == col, ev, 0.0
        )
        out_copy(k).start()
        j = k - 2
        if j >= 0 and j + _NBUF < _CHUNKS:
            out_copy(j).wait()
            waited_out.add(j)
            in_copy(j + _NBUF).start()
    for k in range(_CHUNKS):
        if k not in waited_out:
            out_copy(k).wait()


def kernel(cov, embeddings, noise_scale):
    emb = embeddings[:, :, -1].reshape(_B, 1, _N)
    ns = noise_scale.reshape(1, 1)
    return pl.pallas_call(
        _body,
        in_specs=[
            pl.BlockSpec(memory_space=pltpu.MemorySpace.VMEM),
            pl.BlockSpec(memory_space=pltpu.MemorySpace.VMEM),
            pl.BlockSpec(memory_space=pl.ANY),
        ],
        out_specs=pl.BlockSpec(memory_space=pl.ANY),
        out_shape=jax.ShapeDtypeStruct((_B, _N, _N), jnp.float32),
        scratch_shapes=[
            pltpu.VMEM((_NBUF, _CH, _N), jnp.float32),
            pltpu.SemaphoreType.DMA((_NBUF,)),
            pltpu.SemaphoreType.DMA((_NBUF,)),
        ],
    )(emb, ns, cov)
